# baseline (speedup 1.0000x reference)
import functools

import jax
import jax.numpy as jnp
from jax import lax
from jax.experimental import pallas as pl
from jax.experimental.pallas import tpu as pltpu
from jax.experimental.pallas import tpu_sc as plsc

N = 10000
E = 160000
D = 128
T = 4

NC = 2
NS = 16
LANES = 128
M = 2 * E
PER_TILE_CH = 80
GROUP = 40
PER_TILE = PER_TILE_CH * LANES
MPAD = NC * NS * PER_TILE
ACC_ROWS = 10240
ZROWS = ACC_ROWS // NS

ROW_BLK = 1000


def _idx_kernel(tr_ref, out_ref):
    dst = tr_ref[0:1, :]
    typ = lax.rem(tr_ref[1:2, :], T)
    src = tr_ref[2:3, :]
    out_ref[0:1, :] = typ * N + src
    out_ref[1:2, :] = dst
    out_ref[2:3, :] = (typ + T) * N + dst
    out_ref[3:4, :] = src


def _table_kernel(x_ref, w_ref, b_ref, out_ref):
    x = x_ref[...]
    for t in range(2 * T):
        acc = lax.dot_general(x, w_ref[t], (((1,), (1,)), ((), ())),
                              preferred_element_type=jnp.float32)
        out_ref[t] = acc + b_ref[t]


def _final_kernel(x_ref, w_ref, b_ref, p_ref, out_ref):
    x = x_ref[...]
    acc = lax.dot_general(x, w_ref[...], (((1,), (1,)), ((), ())),
                          preferred_element_type=jnp.float32)
    out_ref[...] = jnp.maximum(acc + b_ref[...] + p_ref[0] + p_ref[1], 0.0)


def _sc_body(table_hbm, gidx_hbm, sidx_hbm, zeros_hbm, out_hbm,
             gidx_v, sidx_v, rows0, rows1, acc_sh, sem0, sem1):
    c = lax.axis_index("c")
    s = lax.axis_index("s")
    bufs = (rows0, rows1)
    sems = (sem0, sem1)
    pltpu.sync_copy(zeros_hbm, acc_sh.at[pl.ds(s * ZROWS, ZROWS)])
    plsc.subcore_barrier()

    for g in range(PER_TILE_CH // GROUP):
        pltpu.sync_copy(gidx_hbm.at[c, s, pl.ds(g * GROUP, GROUP)], gidx_v)
        pltpu.sync_copy(sidx_hbm.at[c, s, pl.ds(g * GROUP, GROUP)], sidx_v)

        def chunk(j, carry):
            pltpu.async_copy(table_hbm.at[gidx_v.at[j]], rows0, sem0).wait()
            pltpu.sync_copy(rows0, acc_sh.at[sidx_v.at[j]], add=True)
            return carry

        lax.fori_loop(0, GROUP, chunk, 0)
    plsc.subcore_barrier()
    pltpu.sync_copy(acc_sh.at[pl.ds(s * ZROWS, ZROWS)],
                    out_hbm.at[c, pl.ds(s * ZROWS, ZROWS)])


@functools.cache
def _sc_messages():
    return pl.kernel(
        _sc_body,
        out_type=jax.ShapeDtypeStruct((NC, ACC_ROWS, D), jnp.float32),
        mesh=plsc.VectorSubcoreMesh(core_axis_name="c", subcore_axis_name="s",
                                    num_cores=NC, num_subcores=NS),
        scratch_types=[
            pltpu.VMEM((GROUP, LANES), jnp.int32),
            pltpu.VMEM((GROUP, LANES), jnp.int32),
            pltpu.VMEM((LANES, D), jnp.float32),
            pltpu.VMEM((LANES, D), jnp.float32),
            pltpu.VMEM_SHARED((ACC_ROWS, D), jnp.float32),
            pltpu.SemaphoreType.DMA,
            pltpu.SemaphoreType.DMA,
        ],
    )


def kernel(input, dependency_triples, W_self, b_self, W, b, Wp, bp):
    x = input.astype(jnp.float32)
    tr = dependency_triples.astype(jnp.int32).T

    idx4 = pl.pallas_call(
        _idx_kernel,
        grid=(E // 16000,),
        in_specs=[pl.BlockSpec((3, 16000), lambda i: (0, i))],
        out_specs=pl.BlockSpec((4, 16000), lambda i: (0, i)),
        out_shape=jax.ShapeDtypeStruct((4, E), jnp.int32),
    )(tr)

    Wc = jnp.concatenate([W, Wp], axis=0)
    bc = jnp.concatenate([b, bp], axis=0).reshape(2 * T, 1, D)

    table = pl.pallas_call(
        _table_kernel,
        grid=(N // ROW_BLK,),
        in_specs=[
            pl.BlockSpec((ROW_BLK, D), lambda i: (i, 0)),
            pl.BlockSpec((2 * T, D, D), lambda i: (0, 0, 0)),
            pl.BlockSpec((2 * T, 1, D), lambda i: (0, 0, 0)),
        ],
        out_specs=pl.BlockSpec((2 * T, ROW_BLK, D), lambda i: (0, i, 0)),
        out_shape=jax.ShapeDtypeStruct((2 * T, N, D), jnp.float32),
    )(x, Wc, bc)

    gather_all = jnp.concatenate([idx4[0], idx4[2]])
    pad_s = N + (jnp.arange(MPAD - M, dtype=jnp.int32) % (ACC_ROWS - N))
    scatter_all = jnp.concatenate([idx4[1], idx4[3], pad_s])
    gather_all = jnp.pad(gather_all, (0, MPAD - M)).reshape(NC, NS,
                                                            PER_TILE_CH, LANES)
    scatter_all = scatter_all.reshape(NC, NS, PER_TILE_CH, LANES)
    zeros = jnp.zeros((ZROWS, D), jnp.float32)

    parts = _sc_messages()(table.reshape(2 * T * N, D), gather_all,
                           scatter_all, zeros)

    out = pl.pallas_call(
        _final_kernel,
        grid=(N // ROW_BLK,),
        in_specs=[
            pl.BlockSpec((ROW_BLK, D), lambda i: (i, 0)),
            pl.BlockSpec((D, D), lambda i: (0, 0)),
            pl.BlockSpec((1, D), lambda i: (0, 0)),
            pl.BlockSpec((NC, ROW_BLK, D), lambda i: (0, i, 0)),
        ],
        out_specs=pl.BlockSpec((ROW_BLK, D), lambda i: (i, 0)),
        out_shape=jax.ShapeDtypeStruct((N, D), jnp.float32),
    )(x, W_self, b_self.reshape(1, D), parts)
    return out


# speedup vs baseline: 1.4802x; 1.4802x over previous
import functools

import jax
import jax.numpy as jnp
from jax import lax
from jax.experimental import pallas as pl
from jax.experimental.pallas import tpu as pltpu
from jax.experimental.pallas import tpu_sc as plsc

N = 10000
E = 160000
D = 128
T = 4

NC = 2
NS = 16
LANES = 128
M = 2 * E
PER_TILE_CH = 79
GROUP = 79
PER_TILE = PER_TILE_CH * LANES
MPAD = NC * NS * PER_TILE
ACC_ROWS = 10240
ZROWS = ACC_ROWS // NS

ROW_BLK = 1000


def _idx_kernel(tr_ref, out_ref):
    dst = tr_ref[0:1, :]
    typ = lax.rem(tr_ref[1:2, :], T)
    src = tr_ref[2:3, :]
    out_ref[0:1, :] = typ * N + src
    out_ref[1:2, :] = dst
    out_ref[2:3, :] = (typ + T) * N + dst
    out_ref[3:4, :] = src


def _table_kernel(x_ref, w_ref, b_ref, out_ref):
    x = x_ref[...]
    for t in range(2 * T):
        acc = lax.dot_general(x, w_ref[t], (((1,), (1,)), ((), ())),
                              preferred_element_type=jnp.float32)
        out_ref[t] = acc + b_ref[t]


def _final_kernel(x_ref, w_ref, b_ref, p_ref, out_ref):
    x = x_ref[...]
    acc = lax.dot_general(x, w_ref[...], (((1,), (1,)), ((), ())),
                          preferred_element_type=jnp.float32)
    out_ref[...] = jnp.maximum(acc + b_ref[...] + p_ref[0] + p_ref[1], 0.0)


def _sc_body(table_hbm, gidx_hbm, sidx_hbm, zeros_hbm, out_hbm,
             gidx_v, sidx_v, rows0, acc_sh, sem0):
    c = lax.axis_index("c")
    s = lax.axis_index("s")
    pltpu.sync_copy(zeros_hbm, acc_sh.at[pl.ds(s * ZROWS, ZROWS)])
    plsc.subcore_barrier()

    for g in range(PER_TILE_CH // GROUP):
        pltpu.sync_copy(gidx_hbm.at[c, s, pl.ds(g * GROUP, GROUP)], gidx_v)
        pltpu.sync_copy(sidx_hbm.at[c, s, pl.ds(g * GROUP, GROUP)], sidx_v)

        def chunk(j, carry):
            pltpu.async_copy(table_hbm.at[gidx_v.at[j]], rows0, sem0).wait()
            pltpu.sync_copy(rows0, acc_sh.at[sidx_v.at[j]], add=True)
            return carry

        lax.fori_loop(0, GROUP, chunk, 0)
    plsc.subcore_barrier()
    pltpu.sync_copy(acc_sh.at[pl.ds(s * ZROWS, ZROWS)],
                    out_hbm.at[c, pl.ds(s * ZROWS, ZROWS)])


@functools.cache
def _sc_messages():
    return pl.kernel(
        _sc_body,
        out_type=jax.ShapeDtypeStruct((NC, ACC_ROWS, D), jnp.float32),
        mesh=plsc.VectorSubcoreMesh(core_axis_name="c", subcore_axis_name="s",
                                    num_cores=NC, num_subcores=NS),
        scratch_types=[
            pltpu.VMEM((GROUP, LANES), jnp.int32),
            pltpu.VMEM((GROUP, LANES), jnp.int32),
            pltpu.VMEM((LANES, D), jnp.float32),
            pltpu.VMEM_SHARED((ACC_ROWS, D), jnp.float32),
            pltpu.SemaphoreType.DMA,
        ],
    )


def kernel(input, dependency_triples, W_self, b_self, W, b, Wp, bp):
    x = input.astype(jnp.float32)
    tr = dependency_triples.astype(jnp.int32).T

    idx4 = pl.pallas_call(
        _idx_kernel,
        grid=(E // 16000,),
        in_specs=[pl.BlockSpec((3, 16000), lambda i: (0, i))],
        out_specs=pl.BlockSpec((4, 16000), lambda i: (0, i)),
        out_shape=jax.ShapeDtypeStruct((4, E), jnp.int32),
    )(tr)

    Wc = jnp.concatenate([W, Wp], axis=0)
    bc = jnp.concatenate([b, bp], axis=0).reshape(2 * T, 1, D)

    table = pl.pallas_call(
        _table_kernel,
        grid=(N // ROW_BLK,),
        in_specs=[
            pl.BlockSpec((ROW_BLK, D), lambda i: (i, 0)),
            pl.BlockSpec((2 * T, D, D), lambda i: (0, 0, 0)),
            pl.BlockSpec((2 * T, 1, D), lambda i: (0, 0, 0)),
        ],
        out_specs=pl.BlockSpec((2 * T, ROW_BLK, D), lambda i: (0, i, 0)),
        out_shape=jax.ShapeDtypeStruct((2 * T, N, D), jnp.float32),
    )(x, Wc, bc)

    gather_all = jnp.concatenate([idx4[0], idx4[2]])
    scatter_all = jnp.concatenate([idx4[1], idx4[3]])
    scatter_all = jnp.pad(scatter_all, (0, MPAD - M), constant_values=N)
    gather_all = jnp.pad(gather_all, (0, MPAD - M)).reshape(
        NS, NC, PER_TILE_CH, LANES).transpose(1, 0, 2, 3)
    scatter_all = scatter_all.reshape(
        NS, NC, PER_TILE_CH, LANES).transpose(1, 0, 2, 3)
    zeros = jnp.zeros((ZROWS, D), jnp.float32)

    parts = _sc_messages()(table.reshape(2 * T * N, D), gather_all,
                           scatter_all, zeros)

    out = pl.pallas_call(
        _final_kernel,
        grid=(N // ROW_BLK,),
        in_specs=[
            pl.BlockSpec((ROW_BLK, D), lambda i: (i, 0)),
            pl.BlockSpec((D, D), lambda i: (0, 0)),
            pl.BlockSpec((1, D), lambda i: (0, 0)),
            pl.BlockSpec((NC, ROW_BLK, D), lambda i: (0, i, 0)),
        ],
        out_specs=pl.BlockSpec((ROW_BLK, D), lambda i: (i, 0)),
        out_shape=jax.ShapeDtypeStruct((N, D), jnp.float32),
    )(x, W_self, b_self.reshape(1, D), parts)
    return out


# speedup vs baseline: 1.5079x; 1.0187x over previous
import functools

import jax
import jax.numpy as jnp
from jax import lax
from jax.experimental import pallas as pl
from jax.experimental.pallas import tpu as pltpu
from jax.experimental.pallas import tpu_sc as plsc

N = 10000
E = 160000
D = 128
T = 4

NC = 2
NS = 16
LANES = 128
M = 2 * E
CH0 = 54
CH1 = 104
CHMAX = 104
MPAD = NS * (CH0 + CH1) * LANES
ACC_ROWS = 10240
ZROWS = ACC_ROWS // NS

ROW_BLK = 1000


def _idx_kernel(tr_ref, out_ref):
    dst = tr_ref[0:1, :]
    typ = lax.rem(tr_ref[1:2, :], T)
    src = tr_ref[2:3, :]
    out_ref[0:1, :] = typ * N + src
    out_ref[1:2, :] = dst
    out_ref[2:3, :] = (typ + T) * N + dst
    out_ref[3:4, :] = src


def _table_kernel(x_ref, w_ref, b_ref, out_ref):
    x = x_ref[...]
    for t in range(2 * T):
        acc = lax.dot_general(x, w_ref[t], (((1,), (1,)), ((), ())),
                              preferred_element_type=jnp.float32)
        out_ref[t] = acc + b_ref[t]


def _final_kernel(x_ref, w_ref, b_ref, p_ref, out_ref):
    x = x_ref[...]
    acc = lax.dot_general(x, w_ref[...], (((1,), (1,)), ((), ())),
                          preferred_element_type=jnp.float32)
    out_ref[...] = jnp.maximum(acc + b_ref[...] + p_ref[0] + p_ref[1], 0.0)


def _sc_body(table_hbm, gidx_hbm, sidx_hbm, zeros_hbm, out_hbm,
             gidx_v, sidx_v, rows0, acc_sh, sem0):
    c = lax.axis_index("c")
    s = lax.axis_index("s")
    pltpu.sync_copy(zeros_hbm, acc_sh.at[pl.ds(s * ZROWS, ZROWS)])
    plsc.subcore_barrier()

    pltpu.sync_copy(gidx_hbm.at[c, s], gidx_v)
    pltpu.sync_copy(sidx_hbm.at[c, s], sidx_v)

    def chunk(j, carry):
        pltpu.async_copy(table_hbm.at[gidx_v.at[j]], rows0, sem0).wait()
        pltpu.sync_copy(rows0, acc_sh.at[sidx_v.at[j]], add=True)
        return carry

    nch = jnp.where(c == 0, CH0, CH1)
    lax.fori_loop(0, nch, chunk, 0)
    plsc.subcore_barrier()
    pltpu.sync_copy(acc_sh.at[pl.ds(s * ZROWS, ZROWS)],
                    out_hbm.at[c, pl.ds(s * ZROWS, ZROWS)])


@functools.cache
def _sc_messages():
    return pl.kernel(
        _sc_body,
        out_type=jax.ShapeDtypeStruct((NC, ACC_ROWS, D), jnp.float32),
        mesh=plsc.VectorSubcoreMesh(core_axis_name="c", subcore_axis_name="s",
                                    num_cores=NC, num_subcores=NS),
        scratch_types=[
            pltpu.VMEM((CHMAX, LANES), jnp.int32),
            pltpu.VMEM((CHMAX, LANES), jnp.int32),
            pltpu.VMEM((LANES, D), jnp.float32),
            pltpu.VMEM_SHARED((ACC_ROWS, D), jnp.float32),
            pltpu.SemaphoreType.DMA,
        ],
    )


def kernel(input, dependency_triples, W_self, b_self, W, b, Wp, bp):
    x = input.astype(jnp.float32)
    tr = dependency_triples.astype(jnp.int32).T

    idx4 = pl.pallas_call(
        _idx_kernel,
        grid=(E // 16000,),
        in_specs=[pl.BlockSpec((3, 16000), lambda i: (0, i))],
        out_specs=pl.BlockSpec((4, 16000), lambda i: (0, i)),
        out_shape=jax.ShapeDtypeStruct((4, E), jnp.int32),
    )(tr)

    Wc = jnp.concatenate([W, Wp], axis=0)
    bc = jnp.concatenate([b, bp], axis=0).reshape(2 * T, 1, D)

    table = pl.pallas_call(
        _table_kernel,
        grid=(N // ROW_BLK,),
        in_specs=[
            pl.BlockSpec((ROW_BLK, D), lambda i: (i, 0)),
            pl.BlockSpec((2 * T, D, D), lambda i: (0, 0, 0)),
            pl.BlockSpec((2 * T, 1, D), lambda i: (0, 0, 0)),
        ],
        out_specs=pl.BlockSpec((2 * T, ROW_BLK, D), lambda i: (0, i, 0)),
        out_shape=jax.ShapeDtypeStruct((2 * T, N, D), jnp.float32),
    )(x, Wc, bc)

    gather_all = jnp.concatenate([idx4[0], idx4[2]])
    scatter_all = jnp.concatenate([idx4[1], idx4[3]])
    scatter_all = jnp.pad(scatter_all, (0, MPAD - M), constant_values=N)
    gather_all = jnp.pad(gather_all, (0, MPAD - M))
    split = NS * CH0 * LANES
    g0 = jnp.pad(gather_all[:split].reshape(NS, CH0, LANES),
                 ((0, 0), (0, CHMAX - CH0), (0, 0)))
    g1 = gather_all[split:].reshape(NS, CH1, LANES)
    gather_all = jnp.stack([g0, g1])
    s0 = jnp.pad(scatter_all[:split].reshape(NS, CH0, LANES),
                 ((0, 0), (0, CHMAX - CH0), (0, 0)), constant_values=N)
    s1 = scatter_all[split:].reshape(NS, CH1, LANES)
    scatter_all = jnp.stack([s0, s1])
    zeros = jnp.zeros((ZROWS, D), jnp.float32)

    parts = _sc_messages()(table.reshape(2 * T * N, D), gather_all,
                           scatter_all, zeros)

    out = pl.pallas_call(
        _final_kernel,
        grid=(N // ROW_BLK,),
        in_specs=[
            pl.BlockSpec((ROW_BLK, D), lambda i: (i, 0)),
            pl.BlockSpec((D, D), lambda i: (0, 0)),
            pl.BlockSpec((1, D), lambda i: (0, 0)),
            pl.BlockSpec((NC, ROW_BLK, D), lambda i: (0, i, 0)),
        ],
        out_specs=pl.BlockSpec((ROW_BLK, D), lambda i: (i, 0)),
        out_shape=jax.ShapeDtypeStruct((N, D), jnp.float32),
    )(x, W_self, b_self.reshape(1, D), parts)
    return out
